# baseline (speedup 1.0000x reference)
import jax
import jax.numpy as jnp
from jax.experimental import pallas as pl
from jax.experimental.pallas import tpu as pltpu

ALLKEY = 32000
HID = 150
GH = 256
B = 16
S = 48


def _gates(gx, gh, h):
    r = jax.nn.sigmoid(gx[:, 0:GH] + gh[:, 0:GH])
    z = jax.nn.sigmoid(gx[:, GH:2 * GH] + gh[:, GH:2 * GH])
    n = jnp.tanh(gx[:, 2 * GH:3 * GH] + r * gh[:, 2 * GH:3 * GH])
    return (1.0 - z) * n + z * h


def _dot(a, b):
    return jnp.dot(a, b, preferred_element_type=jnp.float32)


def _seq_kernel(
    dtf_ref, etf_ref,
    w0f_x_ref, w0f_h_ref, b0f_x_ref, b0f_h_ref,
    w0b_x_ref, w0b_h_ref, b0b_x_ref, b0b_h_ref,
    w1f_x_ref, w1f_h_ref, b1f_x_ref, b1f_h_ref,
    w1b_x_ref, w1b_h_ref, b1b_x_ref, b1b_h_ref,
    wd0_x_ref, wbig_ref, bd0_x_ref, bd0_h_ref,
    bd1_x_ref, bd1_h_ref,
    wo_ref, bo_ref, wq_ref, wk_ref, wv_ref,
    cw_ref, cb_ref,
    endt_ref,
    gx0f_ref, gx0b_ref, x1_ref, gx1f_ref, gx1b_ref,
    ko_ref, q_ref, k_ref, v_ref,
    expa_ref, expal_ref, vw_ref, gxs_ref,
):
    f32 = jnp.float32
    zeros_h = jnp.zeros((B, GH), f32)

    gx0f_ref[...] = _dot(dtf_ref[...], w0f_x_ref[...]) + b0f_x_ref[...]
    gx0b_ref[...] = _dot(dtf_ref[...], w0b_x_ref[...]) + b0b_x_ref[...]

    def enc0_body(t, carry):
        hf, hb = carry
        gxf = gx0f_ref[pl.ds(t * B, B), :]
        ghf = _dot(hf, w0f_h_ref[...]) + b0f_h_ref[...]
        hf = _gates(gxf, ghf, hf)
        x1_ref[t, :, 0:GH] = hf
        tb = (S - 1) - t
        gxb = gx0b_ref[pl.ds(tb * B, B), :]
        ghb = _dot(hb, w0b_h_ref[...]) + b0b_h_ref[...]
        hb = _gates(gxb, ghb, hb)
        x1_ref[tb, :, GH:2 * GH] = hb
        return (hf, hb)

    hf0, hb0 = jax.lax.fori_loop(0, S, enc0_body, (zeros_h, zeros_h))

    x1_flat = x1_ref[...].reshape(S * B, 2 * GH)
    gx1f_ref[...] = _dot(x1_flat, w1f_x_ref[...]) + b1f_x_ref[...]
    gx1b_ref[...] = _dot(x1_flat, w1b_x_ref[...]) + b1b_x_ref[...]

    def enc1_body(t, carry):
        hf, hb = carry
        gxf = gx1f_ref[pl.ds(t * B, B), :]
        hf = _gates(gxf, _dot(hf, w1f_h_ref[...]) + b1f_h_ref[...], hf)
        tb = (S - 1) - t
        gxb = gx1b_ref[pl.ds(tb * B, B), :]
        hb = _gates(gxb, _dot(hb, w1b_h_ref[...]) + b1b_h_ref[...], hb)
        return (hf, hb)

    hf1, hb1 = jax.lax.fori_loop(0, S, enc1_body, (zeros_h, zeros_h))

    h0 = (cw_ref[0, 0] * hf0 + cw_ref[0, 1] * hb0
          + cw_ref[0, 2] * hf1 + cw_ref[0, 3] * hb1 + cb_ref[0])
    h1 = (cw_ref[1, 0] * hf0 + cw_ref[1, 1] * hb0
          + cw_ref[1, 2] * hf1 + cw_ref[1, 3] * hb1 + cb_ref[1])

    ko_ref[...] = _dot(etf_ref[...], wo_ref[...]) + bo_ref[...]
    ko = ko_ref[...]
    q_ref[...] = _dot(ko, wq_ref[...])
    k_ref[...] = _dot(ko, wk_ref[...])
    v_ref[...] = _dot(ko, wv_ref[...])
    vw_ref[...] = _dot(v_ref[...], wd0_x_ref[...])

    for b in range(B):
        qb = q_ref[pl.ds(b * S, S), :]
        kb = k_ref[pl.ds(b * S, S), :]
        ab = jax.lax.dot_general(qb, kb, (((1,), (1,)), ((), ())),
                                 preferred_element_type=f32)
        ea = jnp.exp(ab)
        expa_ref[b] = ea
        expal_ref[:, b, :] = ea

    lane_m = jax.lax.broadcasted_iota(jnp.int32, (B, S), 1)
    b0x = bd0_x_ref[...]
    b0h = bd0_h_ref[...]
    b1x = bd1_x_ref[...]
    b1h = bd1_h_ref[...]

    def fused_parts(h0, h1):
        big = _dot(jnp.concatenate([h0, h1], axis=1), wbig_ref[...])
        gh0 = big[:, 0:3 * GH] + b0h
        gh1 = big[:, 3 * GH:6 * GH] + b1h
        gx1 = big[:, 6 * GH:9 * GH] + b1x
        return gh0, gh1, gx1

    def outer_body(idx, carry):
        h0, h1, den = carry
        den = den + expal_ref[idx]
        scale = jnp.where(lane_m <= idx, 1.0 / den, 0.0)

        for b in range(B):
            pb = expa_ref[b] * scale[b:b + 1, :]
            gxs_ref[:, b, :] = _dot(pb, vw_ref[pl.ds(b * S, S), :]) + b0x

        def inner_body(l, hh):
            h0, h1 = hh
            gh0, gh1, gx1 = fused_parts(h0, h1)
            new_h0 = _gates(gxs_ref[l], gh0, h0)
            new_h1 = _gates(gx1, gh1, h1)
            h1 = jnp.where(l > 0, new_h1, h1)
            return (new_h0, h1)

        h0, h1 = jax.lax.fori_loop(0, idx + 1, inner_body, (h0, h1))
        _, gh1, gx1 = fused_parts(h0, h1)
        h1 = _gates(gx1, gh1, h1)
        endt_ref[idx, :, 0:GH] = h0
        endt_ref[idx, :, GH:2 * GH] = h1
        return (h0, h1, den)

    jax.lax.fori_loop(0, S, outer_body, (h0, h1, jnp.zeros((B, S), f32)))


def _proj_kernel(a_ref, w_ref, b_ref, o_ref):
    o_ref[...] = (_dot(a_ref[...], w_ref[...]) + b_ref[...])


def kernel(x, target, emb, enc0f, enc0b, enc1f, enc1b, dec0, dec1,
           conv_w, conv_b, lin_o, lin_out, linQ, linK, linV):
    f32 = jnp.float32
    dt = emb[x]
    dtf = jnp.swapaxes(dt, 0, 1).reshape(S * B, HID)
    et = jnp.concatenate(
        [jnp.zeros((B, 1, HID), f32), emb[target[:, :S - 1]]], axis=1)
    etf = et.reshape(B * S, HID)

    def gru_w(p):
        wih, whh, bih, bhh = p
        return (wih.T, whh.T, bih.reshape(1, -1), bhh.reshape(1, -1))

    Wo, bo = lin_o
    Wout, bout = lin_out
    WQ, _bQ = linQ
    WK, _bK = linK
    WV, _bV = linV

    wih0, whh0, bih0, bhh0 = dec0
    wih1, whh1, bih1, bhh1 = dec1
    z768 = jnp.zeros((GH, 3 * GH), f32)
    wbig = jnp.concatenate([
        jnp.concatenate([whh0.T, z768, wih1.T], axis=1),
        jnp.concatenate([z768, whh1.T, z768], axis=1),
    ], axis=0)

    args = (
        dtf, etf,
        *gru_w(enc0f), *gru_w(enc0b), *gru_w(enc1f), *gru_w(enc1b),
        wih0.T, wbig, bih0.reshape(1, -1), bhh0.reshape(1, -1),
        bih1.reshape(1, -1), bhh1.reshape(1, -1),
        Wo.T, bo.reshape(1, -1), WQ.T, WK.T, WV.T,
        conv_w[:, :, 0], conv_b,
    )
    n_in = len(args)
    in_specs = [pl.BlockSpec(memory_space=pltpu.VMEM)] * (n_in - 2) + [
        pl.BlockSpec(memory_space=pltpu.SMEM),
        pl.BlockSpec(memory_space=pltpu.SMEM),
    ]

    endt = pl.pallas_call(
        _seq_kernel,
        out_shape=jax.ShapeDtypeStruct((S, B, 2 * GH), f32),
        in_specs=in_specs,
        out_specs=pl.BlockSpec(memory_space=pltpu.VMEM),
        scratch_shapes=[
            pltpu.VMEM((S * B, 3 * GH), f32),
            pltpu.VMEM((S * B, 3 * GH), f32),
            pltpu.VMEM((S, B, 2 * GH), f32),
            pltpu.VMEM((S * B, 3 * GH), f32),
            pltpu.VMEM((S * B, 3 * GH), f32),
            pltpu.VMEM((B * S, 512), f32),
            pltpu.VMEM((B * S, HID), f32),
            pltpu.VMEM((B * S, HID), f32),
            pltpu.VMEM((B * S, HID), f32),
            pltpu.VMEM((B, S, S), f32),
            pltpu.VMEM((S, B, S), f32),
            pltpu.VMEM((B * S, 3 * GH), f32),
            pltpu.VMEM((S, B, 3 * GH), f32),
        ],
        compiler_params=pltpu.CompilerParams(
            vmem_limit_bytes=56 * 1024 * 1024),
        name="seq_decode",
    )(*args)

    e2 = jnp.swapaxes(endt, 0, 1).reshape(B * S, 2 * GH)

    n_tile = 3200
    scinfo = pl.pallas_call(
        _proj_kernel,
        out_shape=jax.ShapeDtypeStruct((B * S, ALLKEY), f32),
        grid=(ALLKEY // n_tile,),
        in_specs=[
            pl.BlockSpec((B * S, 2 * GH), lambda i: (0, 0)),
            pl.BlockSpec((2 * GH, n_tile), lambda i: (0, i)),
            pl.BlockSpec((1, n_tile), lambda i: (0, i)),
        ],
        out_specs=pl.BlockSpec((B * S, n_tile), lambda i: (0, i)),
        compiler_params=pltpu.CompilerParams(
            dimension_semantics=("arbitrary",),
            vmem_limit_bytes=56 * 1024 * 1024),
        name="out_proj",
    )(e2, Wout.T, bout.reshape(1, ALLKEY))

    return scinfo.reshape(B, S, ALLKEY)


# speedup vs baseline: 1.3035x; 1.3035x over previous
import jax
import jax.numpy as jnp
from jax.experimental import pallas as pl
from jax.experimental.pallas import tpu as pltpu

ALLKEY = 32000
HID = 150
GH = 256
B = 16
S = 48


def _gates(gx, gh, h):
    r = jax.nn.sigmoid(gx[:, 0:GH] + gh[:, 0:GH])
    z = jax.nn.sigmoid(gx[:, GH:2 * GH] + gh[:, GH:2 * GH])
    n = jnp.tanh(gx[:, 2 * GH:3 * GH] + r * gh[:, 2 * GH:3 * GH])
    return (1.0 - z) * n + z * h


def _dot(a, b):
    return jnp.dot(a, b, preferred_element_type=jnp.float32)


def _seq_kernel(
    dtf_ref, etf_ref,
    w0f_x_ref, w0f_h_ref, b0f_x_ref, b0f_h_ref,
    w0b_x_ref, w0b_h_ref, b0b_x_ref, b0b_h_ref,
    w1f_x_ref, w1f_h_ref, b1f_x_ref, b1f_h_ref,
    w1b_x_ref, w1b_h_ref, b1b_x_ref, b1b_h_ref,
    wd0_x_ref, w0cat_ref, wbig_ref, bd0_x_ref, bd0_h_ref,
    bd1_x_ref, bd1_h_ref,
    wo_ref, bo_ref, wq_ref, wk_ref, wv_ref,
    cw_ref, cb_ref,
    endt_ref,
    gx0f_ref, gx0b_ref, x1_ref, gx1f_ref, gx1b_ref,
    ko_ref, q_ref, k_ref, v_ref,
    expa_ref, expal_ref, vw_ref, gxs_ref,
):
    f32 = jnp.float32
    zeros_h = jnp.zeros((B, GH), f32)

    gx0f_ref[...] = _dot(dtf_ref[...], w0f_x_ref[...]) + b0f_x_ref[...]
    gx0b_ref[...] = _dot(dtf_ref[...], w0b_x_ref[...]) + b0b_x_ref[...]

    def enc0_body(t, carry):
        hf, hb = carry
        gxf = gx0f_ref[pl.ds(t * B, B), :]
        ghf = _dot(hf, w0f_h_ref[...]) + b0f_h_ref[...]
        hf = _gates(gxf, ghf, hf)
        x1_ref[t, :, 0:GH] = hf
        tb = (S - 1) - t
        gxb = gx0b_ref[pl.ds(tb * B, B), :]
        ghb = _dot(hb, w0b_h_ref[...]) + b0b_h_ref[...]
        hb = _gates(gxb, ghb, hb)
        x1_ref[tb, :, GH:2 * GH] = hb
        return (hf, hb)

    hf0, hb0 = jax.lax.fori_loop(0, S, enc0_body, (zeros_h, zeros_h))

    x1_flat = x1_ref[...].reshape(S * B, 2 * GH)
    gx1f_ref[...] = _dot(x1_flat, w1f_x_ref[...]) + b1f_x_ref[...]
    gx1b_ref[...] = _dot(x1_flat, w1b_x_ref[...]) + b1b_x_ref[...]

    def enc1_body(t, carry):
        hf, hb = carry
        gxf = gx1f_ref[pl.ds(t * B, B), :]
        hf = _gates(gxf, _dot(hf, w1f_h_ref[...]) + b1f_h_ref[...], hf)
        tb = (S - 1) - t
        gxb = gx1b_ref[pl.ds(tb * B, B), :]
        hb = _gates(gxb, _dot(hb, w1b_h_ref[...]) + b1b_h_ref[...], hb)
        return (hf, hb)

    hf1, hb1 = jax.lax.fori_loop(0, S, enc1_body, (zeros_h, zeros_h))

    h0 = (cw_ref[0, 0] * hf0 + cw_ref[0, 1] * hb0
          + cw_ref[0, 2] * hf1 + cw_ref[0, 3] * hb1 + cb_ref[0])
    h1 = (cw_ref[1, 0] * hf0 + cw_ref[1, 1] * hb0
          + cw_ref[1, 2] * hf1 + cw_ref[1, 3] * hb1 + cb_ref[1])

    ko_ref[...] = _dot(etf_ref[...], wo_ref[...]) + bo_ref[...]
    ko = ko_ref[...]
    q_ref[...] = _dot(ko, wq_ref[...])
    k_ref[...] = _dot(ko, wk_ref[...])
    v_ref[...] = _dot(ko, wv_ref[...])
    vw_ref[...] = _dot(v_ref[...], wd0_x_ref[...])

    for b in range(B):
        qb = q_ref[pl.ds(b * S, S), :]
        kb = k_ref[pl.ds(b * S, S), :]
        ab = jax.lax.dot_general(qb, kb, (((1,), (1,)), ((), ())),
                                 preferred_element_type=f32)
        ea = jnp.exp(ab)
        expa_ref[b] = ea
        expal_ref[:, b, :] = ea

    lane_m = jax.lax.broadcasted_iota(jnp.int32, (B, S), 1)
    b0x = bd0_x_ref[...]
    b0h = bd0_h_ref[...]
    b1x = bd1_x_ref[...]
    b1h = bd1_h_ref[...]

    def fused_parts(h0, h1):
        big0 = _dot(h0.astype(jnp.bfloat16), w0cat_ref[...])
        big1 = _dot(h1.astype(jnp.bfloat16), wbig_ref[...])
        gh0 = big0[:, 0:3 * GH] + b0h
        gx1 = big0[:, 3 * GH:6 * GH] + b1x
        gh1 = big1 + b1h
        return gh0, gh1, gx1

    def outer_body(idx, carry):
        h0, h1, den = carry
        den = den + expal_ref[idx]
        scale = jnp.where(lane_m <= idx, 1.0 / den, 0.0)

        for b in range(B):
            pb = expa_ref[b] * scale[b:b + 1, :]
            gxs_ref[:, b, :] = _dot(pb, vw_ref[pl.ds(b * S, S), :]) + b0x

        def inner_body(l, hh):
            h0, h1 = hh
            gh0, gh1, gx1 = fused_parts(h0, h1)
            new_h0 = _gates(gxs_ref[l], gh0, h0)
            new_h1 = _gates(gx1, gh1, h1)
            h1 = jnp.where(l > 0, new_h1, h1)
            return (new_h0, h1)

        h0, h1 = jax.lax.fori_loop(0, idx + 1, inner_body, (h0, h1))
        _, gh1, gx1 = fused_parts(h0, h1)
        h1 = _gates(gx1, gh1, h1)
        endt_ref[idx, :, 0:GH] = h0
        endt_ref[idx, :, GH:2 * GH] = h1
        return (h0, h1, den)

    jax.lax.fori_loop(0, S, outer_body, (h0, h1, jnp.zeros((B, S), f32)))


def _proj_kernel(a_ref, w_ref, b_ref, o_ref):
    o_ref[...] = (_dot(a_ref[...], w_ref[...]) + b_ref[...])


def kernel(x, target, emb, enc0f, enc0b, enc1f, enc1b, dec0, dec1,
           conv_w, conv_b, lin_o, lin_out, linQ, linK, linV):
    f32 = jnp.float32
    dt = emb[x]
    dtf = jnp.swapaxes(dt, 0, 1).reshape(S * B, HID)
    et = jnp.concatenate(
        [jnp.zeros((B, 1, HID), f32), emb[target[:, :S - 1]]], axis=1)
    etf = et.reshape(B * S, HID)

    def gru_w(p):
        wih, whh, bih, bhh = p
        return (wih.T, whh.T, bih.reshape(1, -1), bhh.reshape(1, -1))

    Wo, bo = lin_o
    Wout, bout = lin_out
    WQ, _bQ = linQ
    WK, _bK = linK
    WV, _bV = linV

    wih0, whh0, bih0, bhh0 = dec0
    wih1, whh1, bih1, bhh1 = dec1
    bf16 = jnp.bfloat16
    w0cat = jnp.concatenate([whh0.T, wih1.T], axis=1).astype(bf16)
    wbig = whh1.T.astype(bf16)

    args = (
        dtf, etf,
        *gru_w(enc0f), *gru_w(enc0b), *gru_w(enc1f), *gru_w(enc1b),
        wih0.T, w0cat, wbig, bih0.reshape(1, -1), bhh0.reshape(1, -1),
        bih1.reshape(1, -1), bhh1.reshape(1, -1),
        Wo.T, bo.reshape(1, -1), WQ.T, WK.T, WV.T,
        conv_w[:, :, 0], conv_b,
    )
    n_in = len(args)
    in_specs = [pl.BlockSpec(memory_space=pltpu.VMEM)] * (n_in - 2) + [
        pl.BlockSpec(memory_space=pltpu.SMEM),
        pl.BlockSpec(memory_space=pltpu.SMEM),
    ]

    endt = pl.pallas_call(
        _seq_kernel,
        out_shape=jax.ShapeDtypeStruct((S, B, 2 * GH), f32),
        in_specs=in_specs,
        out_specs=pl.BlockSpec(memory_space=pltpu.VMEM),
        scratch_shapes=[
            pltpu.VMEM((S * B, 3 * GH), f32),
            pltpu.VMEM((S * B, 3 * GH), f32),
            pltpu.VMEM((S, B, 2 * GH), f32),
            pltpu.VMEM((S * B, 3 * GH), f32),
            pltpu.VMEM((S * B, 3 * GH), f32),
            pltpu.VMEM((B * S, 512), f32),
            pltpu.VMEM((B * S, HID), f32),
            pltpu.VMEM((B * S, HID), f32),
            pltpu.VMEM((B * S, HID), f32),
            pltpu.VMEM((B, S, S), f32),
            pltpu.VMEM((S, B, S), f32),
            pltpu.VMEM((B * S, 3 * GH), f32),
            pltpu.VMEM((S, B, 3 * GH), f32),
        ],
        compiler_params=pltpu.CompilerParams(
            vmem_limit_bytes=56 * 1024 * 1024),
        name="seq_decode",
    )(*args)

    e2 = jnp.swapaxes(endt, 0, 1).reshape(B * S, 2 * GH)

    n_tile = 3200
    scinfo = pl.pallas_call(
        _proj_kernel,
        out_shape=jax.ShapeDtypeStruct((B * S, ALLKEY), f32),
        grid=(ALLKEY // n_tile,),
        in_specs=[
            pl.BlockSpec((B * S, 2 * GH), lambda i: (0, 0)),
            pl.BlockSpec((2 * GH, n_tile), lambda i: (0, i)),
            pl.BlockSpec((1, n_tile), lambda i: (0, i)),
        ],
        out_specs=pl.BlockSpec((B * S, n_tile), lambda i: (0, i)),
        compiler_params=pltpu.CompilerParams(
            dimension_semantics=("arbitrary",),
            vmem_limit_bytes=56 * 1024 * 1024),
        name="out_proj",
    )(e2, Wout.T, bout.reshape(1, ALLKEY))

    return scinfo.reshape(B, S, ALLKEY)


# speedup vs baseline: 1.3473x; 1.0336x over previous
import jax
import jax.numpy as jnp
from jax.experimental import pallas as pl
from jax.experimental.pallas import tpu as pltpu

ALLKEY = 32000
HID = 150
GH = 256
B = 16
S = 48


def _gates(gx, gh, h):
    r = jax.nn.sigmoid(gx[:, 0:GH] + gh[:, 0:GH])
    z = jax.nn.sigmoid(gx[:, GH:2 * GH] + gh[:, GH:2 * GH])
    n = jnp.tanh(gx[:, 2 * GH:3 * GH] + r * gh[:, 2 * GH:3 * GH])
    return (1.0 - z) * n + z * h


def _dot(a, b):
    return jnp.dot(a, b, preferred_element_type=jnp.float32)


def _seq_kernel(
    dt_idx_ref, et_idx_ref, emb_ref,
    w0f_x_ref, w0f_h_ref, b0f_x_ref, b0f_h_ref,
    w0b_x_ref, w0b_h_ref, b0b_x_ref, b0b_h_ref,
    w1f_x_ref, w1f_h_ref, b1f_x_ref, b1f_h_ref,
    w1b_x_ref, w1b_h_ref, b1b_x_ref, b1b_h_ref,
    wd0_x_ref, w0cat_ref, wbig_ref, bd0_x_ref, bd0_h_ref,
    bd1_x_ref, bd1_h_ref,
    wo_ref, bo_ref, wq_ref, wk_ref, wv_ref,
    cw_ref, cb_ref,
    endt_ref,
    dtf_ref, etf_ref, sem_dt, sem_et,
    gx0f_ref, gx0b_ref, x1_ref, gx1f_ref, gx1b_ref,
    ko_ref, q_ref, k_ref, v_ref,
    expa_ref, expal_ref, vw_ref, gxs_ref,
):
    f32 = jnp.float32
    zeros_h = jnp.zeros((B, GH), f32)

    for r in range(S * B):
        pltpu.make_async_copy(
            emb_ref.at[dt_idx_ref[r]], dtf_ref.at[r], sem_dt).start()
    for r in range(B * S):
        if r % S == 0:
            etf_ref[pl.ds(r, 1), :] = jnp.zeros((1, HID), f32)
        else:
            pltpu.make_async_copy(
                emb_ref.at[et_idx_ref[r]], etf_ref.at[r], sem_et).start()
    for r in range(S * B):
        pltpu.make_async_copy(
            emb_ref.at[dt_idx_ref[r]], dtf_ref.at[r], sem_dt).wait()

    gx0f_ref[...] = _dot(dtf_ref[...], w0f_x_ref[...]) + b0f_x_ref[...]
    gx0b_ref[...] = _dot(dtf_ref[...], w0b_x_ref[...]) + b0b_x_ref[...]

    def enc0_body(t, carry):
        hf, hb = carry
        gxf = gx0f_ref[pl.ds(t * B, B), :]
        ghf = _dot(hf, w0f_h_ref[...]) + b0f_h_ref[...]
        hf = _gates(gxf, ghf, hf)
        x1_ref[t, :, 0:GH] = hf
        tb = (S - 1) - t
        gxb = gx0b_ref[pl.ds(tb * B, B), :]
        ghb = _dot(hb, w0b_h_ref[...]) + b0b_h_ref[...]
        hb = _gates(gxb, ghb, hb)
        x1_ref[tb, :, GH:2 * GH] = hb
        return (hf, hb)

    hf0, hb0 = jax.lax.fori_loop(0, S, enc0_body, (zeros_h, zeros_h))

    x1_flat = x1_ref[...].reshape(S * B, 2 * GH)
    gx1f_ref[...] = _dot(x1_flat, w1f_x_ref[...]) + b1f_x_ref[...]
    gx1b_ref[...] = _dot(x1_flat, w1b_x_ref[...]) + b1b_x_ref[...]

    def enc1_body(t, carry):
        hf, hb = carry
        gxf = gx1f_ref[pl.ds(t * B, B), :]
        hf = _gates(gxf, _dot(hf, w1f_h_ref[...]) + b1f_h_ref[...], hf)
        tb = (S - 1) - t
        gxb = gx1b_ref[pl.ds(tb * B, B), :]
        hb = _gates(gxb, _dot(hb, w1b_h_ref[...]) + b1b_h_ref[...], hb)
        return (hf, hb)

    hf1, hb1 = jax.lax.fori_loop(0, S, enc1_body, (zeros_h, zeros_h))

    h0 = (cw_ref[0, 0] * hf0 + cw_ref[0, 1] * hb0
          + cw_ref[0, 2] * hf1 + cw_ref[0, 3] * hb1 + cb_ref[0])
    h1 = (cw_ref[1, 0] * hf0 + cw_ref[1, 1] * hb0
          + cw_ref[1, 2] * hf1 + cw_ref[1, 3] * hb1 + cb_ref[1])

    for r in range(B * S):
        if r % S != 0:
            pltpu.make_async_copy(
                emb_ref.at[et_idx_ref[r]], etf_ref.at[r], sem_et).wait()
    ko_ref[...] = _dot(etf_ref[...], wo_ref[...]) + bo_ref[...]
    ko = ko_ref[...]
    q_ref[...] = _dot(ko, wq_ref[...])
    k_ref[...] = _dot(ko, wk_ref[...])
    v_ref[...] = _dot(ko, wv_ref[...])
    vw_ref[...] = _dot(v_ref[...], wd0_x_ref[...])

    for b in range(B):
        qb = q_ref[pl.ds(b * S, S), :]
        kb = k_ref[pl.ds(b * S, S), :]
        ab = jax.lax.dot_general(qb, kb, (((1,), (1,)), ((), ())),
                                 preferred_element_type=f32)
        ea = jnp.exp(ab)
        expa_ref[b] = ea
        expal_ref[:, b, :] = ea

    lane_m = jax.lax.broadcasted_iota(jnp.int32, (B, S), 1)
    b0x = bd0_x_ref[...]
    b0h = bd0_h_ref[...]
    b1x = bd1_x_ref[...]
    b1h = bd1_h_ref[...]

    def fused_parts(h0, h1):
        big0 = _dot(h0.astype(jnp.bfloat16), w0cat_ref[...])
        big1 = _dot(h1.astype(jnp.bfloat16), wbig_ref[...])
        gh0 = big0[:, 0:3 * GH] + b0h
        gx1 = big0[:, 3 * GH:6 * GH] + b1x
        gh1 = big1 + b1h
        return gh0, gh1, gx1

    def outer_body(idx, carry):
        h0, h1, den = carry
        den = den + expal_ref[idx]
        scale = jnp.where(lane_m <= idx, 1.0 / den, 0.0)

        for b in range(B):
            pb = expa_ref[b] * scale[b:b + 1, :]
            gxs_ref[:, b, :] = _dot(pb, vw_ref[pl.ds(b * S, S), :]) + b0x

        def inner_body(l, hh):
            h0, h1 = hh
            gh0, gh1, gx1 = fused_parts(h0, h1)
            new_h0 = _gates(gxs_ref[l], gh0, h0)
            new_h1 = _gates(gx1, gh1, h1)
            h1 = jnp.where(l > 0, new_h1, h1)
            return (new_h0, h1)

        h0, h1 = jax.lax.fori_loop(0, idx + 1, inner_body, (h0, h1))
        _, gh1, gx1 = fused_parts(h0, h1)
        h1 = _gates(gx1, gh1, h1)
        endt_ref[idx, :, 0:GH] = h0
        endt_ref[idx, :, GH:2 * GH] = h1
        return (h0, h1, den)

    jax.lax.fori_loop(0, S, outer_body, (h0, h1, jnp.zeros((B, S), f32)))


def _proj_kernel(a_ref, w_ref, b_ref, o_ref, a2_ref):
    @pl.when(pl.program_id(0) == 0)
    def _():
        a2_ref[...] = jnp.swapaxes(a_ref[...], 0, 1).reshape(B * S, 2 * GH)

    o_ref[...] = (_dot(a2_ref[...], w_ref[...]) + b_ref[...])


def kernel(x, target, emb, enc0f, enc0b, enc1f, enc1b, dec0, dec1,
           conv_w, conv_b, lin_o, lin_out, linQ, linK, linV):
    f32 = jnp.float32
    dt_idx = x.T.reshape(S * B)
    et_idx = jnp.concatenate(
        [jnp.zeros((B, 1), jnp.int32), target[:, :S - 1]], axis=1
    ).reshape(B * S)

    def gru_w(p):
        wih, whh, bih, bhh = p
        return (wih.T, whh.T, bih.reshape(1, -1), bhh.reshape(1, -1))

    Wo, bo = lin_o
    Wout, bout = lin_out
    WQ, _bQ = linQ
    WK, _bK = linK
    WV, _bV = linV

    wih0, whh0, bih0, bhh0 = dec0
    wih1, whh1, bih1, bhh1 = dec1
    bf16 = jnp.bfloat16
    w0cat = jnp.concatenate([whh0.T, wih1.T], axis=1).astype(bf16)
    wbig = whh1.T.astype(bf16)

    args = (
        dt_idx, et_idx, emb,
        *gru_w(enc0f), *gru_w(enc0b), *gru_w(enc1f), *gru_w(enc1b),
        wih0.T, w0cat, wbig, bih0.reshape(1, -1), bhh0.reshape(1, -1),
        bih1.reshape(1, -1), bhh1.reshape(1, -1),
        Wo.T, bo.reshape(1, -1), WQ.T, WK.T, WV.T,
        conv_w[:, :, 0], conv_b,
    )
    n_in = len(args)
    in_specs = [
        pl.BlockSpec(memory_space=pltpu.SMEM),
        pl.BlockSpec(memory_space=pltpu.SMEM),
        pl.BlockSpec(memory_space=pl.ANY),
    ] + [pl.BlockSpec(memory_space=pltpu.VMEM)] * (n_in - 5) + [
        pl.BlockSpec(memory_space=pltpu.SMEM),
        pl.BlockSpec(memory_space=pltpu.SMEM),
    ]

    endt = pl.pallas_call(
        _seq_kernel,
        out_shape=jax.ShapeDtypeStruct((S, B, 2 * GH), f32),
        in_specs=in_specs,
        out_specs=pl.BlockSpec(memory_space=pltpu.VMEM),
        scratch_shapes=[
            pltpu.VMEM((S * B, HID), f32),
            pltpu.VMEM((B * S, HID), f32),
            pltpu.SemaphoreType.DMA,
            pltpu.SemaphoreType.DMA,
            pltpu.VMEM((S * B, 3 * GH), f32),
            pltpu.VMEM((S * B, 3 * GH), f32),
            pltpu.VMEM((S, B, 2 * GH), f32),
            pltpu.VMEM((S * B, 3 * GH), f32),
            pltpu.VMEM((S * B, 3 * GH), f32),
            pltpu.VMEM((B * S, 512), f32),
            pltpu.VMEM((B * S, HID), f32),
            pltpu.VMEM((B * S, HID), f32),
            pltpu.VMEM((B * S, HID), f32),
            pltpu.VMEM((B, S, S), f32),
            pltpu.VMEM((S, B, S), f32),
            pltpu.VMEM((B * S, 3 * GH), f32),
            pltpu.VMEM((S, B, 3 * GH), f32),
        ],
        compiler_params=pltpu.CompilerParams(
            vmem_limit_bytes=56 * 1024 * 1024),
        name="seq_decode",
    )(*args)

    n_tile = 3200
    scinfo = pl.pallas_call(
        _proj_kernel,
        out_shape=jax.ShapeDtypeStruct((B * S, ALLKEY), f32),
        grid=(ALLKEY // n_tile,),
        in_specs=[
            pl.BlockSpec((S, B, 2 * GH), lambda i: (0, 0, 0)),
            pl.BlockSpec((2 * GH, n_tile), lambda i: (0, i)),
            pl.BlockSpec((1, n_tile), lambda i: (0, i)),
        ],
        out_specs=pl.BlockSpec((B * S, n_tile), lambda i: (0, i)),
        scratch_shapes=[pltpu.VMEM((B * S, 2 * GH), f32)],
        compiler_params=pltpu.CompilerParams(
            dimension_semantics=("arbitrary",),
            vmem_limit_bytes=56 * 1024 * 1024),
        name="out_proj",
    )(endt, Wout.T, bout.reshape(1, ALLKEY))

    return scinfo.reshape(B, S, ALLKEY)


# speedup vs baseline: 1.4178x; 1.0523x over previous
import jax
import jax.numpy as jnp
from jax.experimental import pallas as pl
from jax.experimental.pallas import tpu as pltpu

ALLKEY = 32000
HID = 150
GH = 256
B = 16
S = 48


def _gates(gx, gh, h):
    r = jax.nn.sigmoid(gx[:, 0:GH] + gh[:, 0:GH])
    z = jax.nn.sigmoid(gx[:, GH:2 * GH] + gh[:, GH:2 * GH])
    n = jnp.tanh(gx[:, 2 * GH:3 * GH] + r * gh[:, 2 * GH:3 * GH])
    return (1.0 - z) * n + z * h


def _dot(a, b):
    return jnp.dot(a, b, preferred_element_type=jnp.float32)


def _seq_kernel(
    dt_idx_ref, et_idx_ref, emb_ref,
    w0f_x_ref, w0f_h_ref, b0f_x_ref, b0f_h_ref,
    w0b_x_ref, w0b_h_ref, b0b_x_ref, b0b_h_ref,
    w1f_x_ref, w1f_h_ref, b1f_x_ref, b1f_h_ref,
    w1b_x_ref, w1b_h_ref, b1b_x_ref, b1b_h_ref,
    wd0_x_ref, w0cat_ref, wbig_ref, bd0_x_ref, bd0_h_ref,
    bd1_x_ref, bd1_h_ref,
    wo_ref, bo_ref, wq_ref, wk_ref, wv_ref,
    cw_ref, cb_ref,
    endt_ref,
    dtf_ref, etf_ref, sem_dt, sem_et,
    gx0f_ref, gx0b_ref, x1_ref, gx1f_ref, gx1b_ref,
    ko_ref, q_ref, k_ref, v_ref,
    expa_ref, expal_ref, vw_ref, gxs_ref,
):
    f32 = jnp.float32
    zeros_h = jnp.zeros((B, GH), f32)

    for r in range(S * B):
        pltpu.make_async_copy(
            emb_ref.at[dt_idx_ref[r]], dtf_ref.at[r], sem_dt).start()
    for r in range(B * S):
        if r % S == 0:
            etf_ref[pl.ds(r, 1), :] = jnp.zeros((1, HID), f32)
        else:
            pltpu.make_async_copy(
                emb_ref.at[et_idx_ref[r]], etf_ref.at[r], sem_et).start()
    for r in range(S * B):
        pltpu.make_async_copy(
            emb_ref.at[dt_idx_ref[r]], dtf_ref.at[r], sem_dt).wait()

    gx0f_ref[...] = _dot(dtf_ref[...], w0f_x_ref[...]) + b0f_x_ref[...]
    gx0b_ref[...] = _dot(dtf_ref[...], w0b_x_ref[...]) + b0b_x_ref[...]

    def enc0_body(t, carry):
        hf, hb = carry
        gxf = gx0f_ref[pl.ds(t * B, B), :]
        ghf = _dot(hf, w0f_h_ref[...]) + b0f_h_ref[...]
        hf = _gates(gxf, ghf, hf)
        x1_ref[t, :, 0:GH] = hf
        tb = (S - 1) - t
        gxb = gx0b_ref[pl.ds(tb * B, B), :]
        ghb = _dot(hb, w0b_h_ref[...]) + b0b_h_ref[...]
        hb = _gates(gxb, ghb, hb)
        x1_ref[tb, :, GH:2 * GH] = hb
        return (hf, hb)

    hf0, hb0 = jax.lax.fori_loop(0, S, enc0_body, (zeros_h, zeros_h))

    x1_flat = x1_ref[...].reshape(S * B, 2 * GH)
    gx1f_ref[...] = _dot(x1_flat, w1f_x_ref[...]) + b1f_x_ref[...]
    gx1b_ref[...] = _dot(x1_flat, w1b_x_ref[...]) + b1b_x_ref[...]

    def enc1_body(t, carry):
        hf, hb = carry
        gxf = gx1f_ref[pl.ds(t * B, B), :]
        hf = _gates(gxf, _dot(hf, w1f_h_ref[...]) + b1f_h_ref[...], hf)
        tb = (S - 1) - t
        gxb = gx1b_ref[pl.ds(tb * B, B), :]
        hb = _gates(gxb, _dot(hb, w1b_h_ref[...]) + b1b_h_ref[...], hb)
        return (hf, hb)

    hf1, hb1 = jax.lax.fori_loop(0, S, enc1_body, (zeros_h, zeros_h))

    h0 = (cw_ref[0, 0] * hf0 + cw_ref[0, 1] * hb0
          + cw_ref[0, 2] * hf1 + cw_ref[0, 3] * hb1 + cb_ref[0])
    h1 = (cw_ref[1, 0] * hf0 + cw_ref[1, 1] * hb0
          + cw_ref[1, 2] * hf1 + cw_ref[1, 3] * hb1 + cb_ref[1])

    for r in range(B * S):
        if r % S != 0:
            pltpu.make_async_copy(
                emb_ref.at[et_idx_ref[r]], etf_ref.at[r], sem_et).wait()
    ko_ref[...] = _dot(etf_ref[...], wo_ref[...]) + bo_ref[...]
    ko = ko_ref[...]
    q_ref[...] = _dot(ko, wq_ref[...])
    k_ref[...] = _dot(ko, wk_ref[...])
    v_ref[...] = _dot(ko, wv_ref[...])
    vw_ref[...] = _dot(v_ref[...], wd0_x_ref[...])

    for b in range(B):
        qb = q_ref[pl.ds(b * S, S), :]
        kb = k_ref[pl.ds(b * S, S), :]
        ab = jax.lax.dot_general(qb, kb, (((1,), (1,)), ((), ())),
                                 preferred_element_type=f32)
        ea = jnp.exp(ab)
        expa_ref[b] = ea
        expal_ref[:, b, :] = ea

    lane_m = jax.lax.broadcasted_iota(jnp.int32, (B, S), 1)
    b0x = bd0_x_ref[...]
    b0h = bd0_h_ref[...]
    b1x = bd1_x_ref[...]
    b1h = bd1_h_ref[...]

    def fused_parts(h0, h1):
        big0 = _dot(h0.astype(jnp.bfloat16), w0cat_ref[...])
        big1 = _dot(h1.astype(jnp.bfloat16), wbig_ref[...])
        gh0 = big0[:, 0:3 * GH] + b0h
        gx1 = big0[:, 3 * GH:6 * GH] + b1x
        gh1 = big1 + b1h
        return gh0, gh1, gx1

    def outer_body(idx, carry):
        h0, h1, den = carry
        den = den + expal_ref[idx]
        scale = jnp.where(lane_m <= idx, 1.0 / den, 0.0)

        for b in range(B):
            pb = expa_ref[b] * scale[b:b + 1, :]
            gxs_ref[:, b, :] = _dot(pb, vw_ref[pl.ds(b * S, S), :]) + b0x

        def inner_body(l, hh):
            h0, h1 = hh
            gh0, gh1, gx1 = fused_parts(h0, h1)
            new_h0 = _gates(gxs_ref[l], gh0, h0)
            new_h1 = _gates(gx1, gh1, h1)
            h1 = jnp.where(l > 0, new_h1, h1)
            return (new_h0, h1)

        h0, h1 = jax.lax.fori_loop(0, idx + 1, inner_body, (h0, h1))
        _, gh1, gx1 = fused_parts(h0, h1)
        h1 = _gates(gx1, gh1, h1)
        endt_ref[idx, :, 0:GH] = h0
        endt_ref[idx, :, GH:2 * GH] = h1
        return (h0, h1, den)

    jax.lax.fori_loop(0, S, outer_body, (h0, h1, jnp.zeros((B, S), f32)))


def _proj_kernel(a_ref, w_ref, b_ref, o_ref, a2_ref):
    @pl.when(pl.program_id(0) == 0)
    def _():
        a2_ref[...] = jnp.swapaxes(a_ref[...], 0, 1).reshape(B * S, 2 * GH)

    o_ref[...] = jax.lax.dot_general(
        a2_ref[...], w_ref[...], (((1,), (1,)), ((), ())),
        preferred_element_type=jnp.float32) + b_ref[...]


def kernel(x, target, emb, enc0f, enc0b, enc1f, enc1b, dec0, dec1,
           conv_w, conv_b, lin_o, lin_out, linQ, linK, linV):
    f32 = jnp.float32
    dt_idx = x.T.reshape(S * B)
    et_idx = jnp.concatenate(
        [jnp.zeros((B, 1), jnp.int32), target[:, :S - 1]], axis=1
    ).reshape(B * S)

    def gru_w(p):
        wih, whh, bih, bhh = p
        return (wih.T, whh.T, bih.reshape(1, -1), bhh.reshape(1, -1))

    Wo, bo = lin_o
    Wout, bout = lin_out
    WQ, _bQ = linQ
    WK, _bK = linK
    WV, _bV = linV

    wih0, whh0, bih0, bhh0 = dec0
    wih1, whh1, bih1, bhh1 = dec1
    bf16 = jnp.bfloat16
    w0cat = jnp.concatenate([whh0.T, wih1.T], axis=1).astype(bf16)
    wbig = whh1.T.astype(bf16)

    args = (
        dt_idx, et_idx, emb,
        *gru_w(enc0f), *gru_w(enc0b), *gru_w(enc1f), *gru_w(enc1b),
        wih0.T, w0cat, wbig, bih0.reshape(1, -1), bhh0.reshape(1, -1),
        bih1.reshape(1, -1), bhh1.reshape(1, -1),
        Wo.T, bo.reshape(1, -1), WQ.T, WK.T, WV.T,
        conv_w[:, :, 0], conv_b,
    )
    n_in = len(args)
    in_specs = [
        pl.BlockSpec(memory_space=pltpu.SMEM),
        pl.BlockSpec(memory_space=pltpu.SMEM),
        pl.BlockSpec(memory_space=pl.ANY),
    ] + [pl.BlockSpec(memory_space=pltpu.VMEM)] * (n_in - 5) + [
        pl.BlockSpec(memory_space=pltpu.SMEM),
        pl.BlockSpec(memory_space=pltpu.SMEM),
    ]

    endt = pl.pallas_call(
        _seq_kernel,
        out_shape=jax.ShapeDtypeStruct((S, B, 2 * GH), f32),
        in_specs=in_specs,
        out_specs=pl.BlockSpec(memory_space=pltpu.VMEM),
        scratch_shapes=[
            pltpu.VMEM((S * B, HID), f32),
            pltpu.VMEM((B * S, HID), f32),
            pltpu.SemaphoreType.DMA,
            pltpu.SemaphoreType.DMA,
            pltpu.VMEM((S * B, 3 * GH), f32),
            pltpu.VMEM((S * B, 3 * GH), f32),
            pltpu.VMEM((S, B, 2 * GH), f32),
            pltpu.VMEM((S * B, 3 * GH), f32),
            pltpu.VMEM((S * B, 3 * GH), f32),
            pltpu.VMEM((B * S, 512), f32),
            pltpu.VMEM((B * S, HID), f32),
            pltpu.VMEM((B * S, HID), f32),
            pltpu.VMEM((B * S, HID), f32),
            pltpu.VMEM((B, S, S), f32),
            pltpu.VMEM((S, B, S), f32),
            pltpu.VMEM((B * S, 3 * GH), f32),
            pltpu.VMEM((S, B, 3 * GH), f32),
        ],
        compiler_params=pltpu.CompilerParams(
            vmem_limit_bytes=56 * 1024 * 1024),
        name="seq_decode",
    )(*args)

    n_tile = 3200
    scinfo = pl.pallas_call(
        _proj_kernel,
        out_shape=jax.ShapeDtypeStruct((B * S, ALLKEY), f32),
        grid=(ALLKEY // n_tile,),
        in_specs=[
            pl.BlockSpec((S, B, 2 * GH), lambda i: (0, 0, 0)),
            pl.BlockSpec((n_tile, 2 * GH), lambda i: (i, 0)),
            pl.BlockSpec((1, n_tile), lambda i: (0, i)),
        ],
        out_specs=pl.BlockSpec((B * S, n_tile), lambda i: (0, i)),
        scratch_shapes=[pltpu.VMEM((B * S, 2 * GH), f32)],
        compiler_params=pltpu.CompilerParams(
            dimension_semantics=("arbitrary",),
            vmem_limit_bytes=56 * 1024 * 1024),
        name="out_proj",
    )(endt, Wout, bout.reshape(1, ALLKEY))

    return scinfo.reshape(B, S, ALLKEY)


# speedup vs baseline: 1.4181x; 1.0002x over previous
import jax
import jax.numpy as jnp
from jax.experimental import pallas as pl
from jax.experimental.pallas import tpu as pltpu

ALLKEY = 32000
HID = 150
GH = 256
B = 16
S = 48


def _gates(gx, gh, h):
    r = jax.nn.sigmoid(gx[:, 0:GH] + gh[:, 0:GH])
    z = jax.nn.sigmoid(gx[:, GH:2 * GH] + gh[:, GH:2 * GH])
    n = jnp.tanh(gx[:, 2 * GH:3 * GH] + r * gh[:, 2 * GH:3 * GH])
    return (1.0 - z) * n + z * h


def _dot(a, b):
    return jnp.dot(a, b, preferred_element_type=jnp.float32)


def _seq_kernel(
    dt_idx_ref, et_idx_ref, emb_ref,
    w0f_x_ref, w0f_h_ref, b0f_x_ref, b0f_h_ref,
    w0b_x_ref, w0b_h_ref, b0b_x_ref, b0b_h_ref,
    w1f_x_ref, w1f_h_ref, b1f_x_ref, b1f_h_ref,
    w1b_x_ref, w1b_h_ref, b1b_x_ref, b1b_h_ref,
    wd0_x_ref, w0cat_ref, wbig_ref, bd0_x_ref, bd0_h_ref,
    bd1_x_ref, bd1_h_ref,
    wo_ref, bo_ref, wq_ref, wk_ref, wv_ref,
    cw_ref, cb_ref,
    endt_ref,
    dtf_ref, etf_ref, sem_dt, sem_et,
    gx0f_ref, gx0b_ref, x1_ref, gx1f_ref, gx1b_ref,
    ko_ref, q_ref, k_ref, v_ref,
    expa_ref, expal_ref, vw_ref, gxs_ref,
):
    f32 = jnp.float32
    zeros_h = jnp.zeros((B, GH), f32)

    for r in range(S * B):
        t, b = r // B, r % B
        pltpu.make_async_copy(
            emb_ref.at[dt_idx_ref[b * S + t]], dtf_ref.at[r], sem_dt).start()
    for r in range(B * S):
        if r % S == 0:
            etf_ref[pl.ds(r, 1), :] = jnp.zeros((1, HID), f32)
        else:
            pltpu.make_async_copy(
                emb_ref.at[et_idx_ref[r - 1]], etf_ref.at[r], sem_et).start()
    for r in range(S * B):
        t, b = r // B, r % B
        pltpu.make_async_copy(
            emb_ref.at[dt_idx_ref[b * S + t]], dtf_ref.at[r], sem_dt).wait()

    gx0f_ref[...] = _dot(dtf_ref[...], w0f_x_ref[...]) + b0f_x_ref[...]
    gx0b_ref[...] = _dot(dtf_ref[...], w0b_x_ref[...]) + b0b_x_ref[...]

    def enc0_body(t, carry):
        hf, hb = carry
        gxf = gx0f_ref[pl.ds(t * B, B), :]
        ghf = _dot(hf, w0f_h_ref[...]) + b0f_h_ref[...]
        hf = _gates(gxf, ghf, hf)
        x1_ref[t, :, 0:GH] = hf
        tb = (S - 1) - t
        gxb = gx0b_ref[pl.ds(tb * B, B), :]
        ghb = _dot(hb, w0b_h_ref[...]) + b0b_h_ref[...]
        hb = _gates(gxb, ghb, hb)
        x1_ref[tb, :, GH:2 * GH] = hb
        return (hf, hb)

    hf0, hb0 = jax.lax.fori_loop(0, S, enc0_body, (zeros_h, zeros_h))

    x1_flat = x1_ref[...].reshape(S * B, 2 * GH)
    gx1f_ref[...] = _dot(x1_flat, w1f_x_ref[...]) + b1f_x_ref[...]
    gx1b_ref[...] = _dot(x1_flat, w1b_x_ref[...]) + b1b_x_ref[...]

    def enc1_body(t, carry):
        hf, hb = carry
        gxf = gx1f_ref[pl.ds(t * B, B), :]
        hf = _gates(gxf, _dot(hf, w1f_h_ref[...]) + b1f_h_ref[...], hf)
        tb = (S - 1) - t
        gxb = gx1b_ref[pl.ds(tb * B, B), :]
        hb = _gates(gxb, _dot(hb, w1b_h_ref[...]) + b1b_h_ref[...], hb)
        return (hf, hb)

    hf1, hb1 = jax.lax.fori_loop(0, S, enc1_body, (zeros_h, zeros_h))

    h0 = (cw_ref[0, 0] * hf0 + cw_ref[0, 1] * hb0
          + cw_ref[0, 2] * hf1 + cw_ref[0, 3] * hb1 + cb_ref[0])
    h1 = (cw_ref[1, 0] * hf0 + cw_ref[1, 1] * hb0
          + cw_ref[1, 2] * hf1 + cw_ref[1, 3] * hb1 + cb_ref[1])

    for r in range(B * S):
        if r % S != 0:
            pltpu.make_async_copy(
                emb_ref.at[et_idx_ref[r - 1]], etf_ref.at[r], sem_et).wait()
    ko_ref[...] = _dot(etf_ref[...], wo_ref[...]) + bo_ref[...]
    ko = ko_ref[...]
    q_ref[...] = _dot(ko, wq_ref[...])
    k_ref[...] = _dot(ko, wk_ref[...])
    v_ref[...] = _dot(ko, wv_ref[...])
    vw_ref[...] = _dot(v_ref[...], wd0_x_ref[...])

    for b in range(B):
        qb = q_ref[pl.ds(b * S, S), :]
        kb = k_ref[pl.ds(b * S, S), :]
        ab = jax.lax.dot_general(qb, kb, (((1,), (1,)), ((), ())),
                                 preferred_element_type=f32)
        ea = jnp.exp(ab)
        expa_ref[b] = ea
        expal_ref[:, b, :] = ea

    lane_m = jax.lax.broadcasted_iota(jnp.int32, (B, S), 1)
    b0x = bd0_x_ref[...]
    b0h = bd0_h_ref[...]
    b1x = bd1_x_ref[...]
    b1h = bd1_h_ref[...]

    def fused_parts(h0, h1):
        big0 = _dot(h0.astype(jnp.bfloat16), w0cat_ref[...])
        big1 = _dot(h1.astype(jnp.bfloat16), wbig_ref[...])
        gh0 = big0[:, 0:3 * GH] + b0h
        gx1 = big0[:, 3 * GH:6 * GH] + b1x
        gh1 = big1 + b1h
        return gh0, gh1, gx1

    def outer_body(idx, carry):
        h0, h1, den = carry
        den = den + expal_ref[idx]
        scale = jnp.where(lane_m <= idx, 1.0 / den, 0.0)

        for b in range(B):
            pb = expa_ref[b] * scale[b:b + 1, :]
            gxs_ref[:, b, :] = _dot(pb, vw_ref[pl.ds(b * S, S), :]) + b0x

        def inner_body(l, hh):
            h0, h1 = hh
            gh0, gh1, gx1 = fused_parts(h0, h1)
            new_h0 = _gates(gxs_ref[l], gh0, h0)
            new_h1 = _gates(gx1, gh1, h1)
            h1 = jnp.where(l > 0, new_h1, h1)
            return (new_h0, h1)

        h0, h1 = jax.lax.fori_loop(0, idx + 1, inner_body, (h0, h1))
        _, gh1, gx1 = fused_parts(h0, h1)
        h1 = _gates(gx1, gh1, h1)
        endt_ref[idx, :, 0:GH] = h0
        endt_ref[idx, :, GH:2 * GH] = h1
        return (h0, h1, den)

    jax.lax.fori_loop(0, S, outer_body, (h0, h1, jnp.zeros((B, S), f32)))


def _proj_kernel(a_ref, w_ref, b_ref, o_ref, a2_ref):
    @pl.when(pl.program_id(0) == 0)
    def _():
        a2_ref[...] = jnp.swapaxes(a_ref[...], 0, 1).reshape(B * S, 2 * GH)

    o_ref[...] = jax.lax.dot_general(
        a2_ref[...], w_ref[...], (((1,), (1,)), ((), ())),
        preferred_element_type=jnp.float32) + b_ref[...]


def kernel(x, target, emb, enc0f, enc0b, enc1f, enc1b, dec0, dec1,
           conv_w, conv_b, lin_o, lin_out, linQ, linK, linV):
    f32 = jnp.float32
    dt_idx = x.reshape(B * S)
    et_idx = target.reshape(B * S)

    def gru_w(p):
        wih, whh, bih, bhh = p
        return (wih.T, whh.T, bih.reshape(1, -1), bhh.reshape(1, -1))

    Wo, bo = lin_o
    Wout, bout = lin_out
    WQ, _bQ = linQ
    WK, _bK = linK
    WV, _bV = linV

    wih0, whh0, bih0, bhh0 = dec0
    wih1, whh1, bih1, bhh1 = dec1
    bf16 = jnp.bfloat16
    w0cat = jnp.concatenate([whh0.T, wih1.T], axis=1).astype(bf16)
    wbig = whh1.T.astype(bf16)

    args = (
        dt_idx, et_idx, emb,
        *gru_w(enc0f), *gru_w(enc0b), *gru_w(enc1f), *gru_w(enc1b),
        wih0.T, w0cat, wbig, bih0.reshape(1, -1), bhh0.reshape(1, -1),
        bih1.reshape(1, -1), bhh1.reshape(1, -1),
        Wo.T, bo.reshape(1, -1), WQ.T, WK.T, WV.T,
        conv_w[:, :, 0], conv_b,
    )
    n_in = len(args)
    in_specs = [
        pl.BlockSpec(memory_space=pltpu.SMEM),
        pl.BlockSpec(memory_space=pltpu.SMEM),
        pl.BlockSpec(memory_space=pl.ANY),
    ] + [pl.BlockSpec(memory_space=pltpu.VMEM)] * (n_in - 5) + [
        pl.BlockSpec(memory_space=pltpu.SMEM),
        pl.BlockSpec(memory_space=pltpu.SMEM),
    ]

    endt = pl.pallas_call(
        _seq_kernel,
        out_shape=jax.ShapeDtypeStruct((S, B, 2 * GH), f32),
        in_specs=in_specs,
        out_specs=pl.BlockSpec(memory_space=pltpu.VMEM),
        scratch_shapes=[
            pltpu.VMEM((S * B, HID), f32),
            pltpu.VMEM((B * S, HID), f32),
            pltpu.SemaphoreType.DMA,
            pltpu.SemaphoreType.DMA,
            pltpu.VMEM((S * B, 3 * GH), f32),
            pltpu.VMEM((S * B, 3 * GH), f32),
            pltpu.VMEM((S, B, 2 * GH), f32),
            pltpu.VMEM((S * B, 3 * GH), f32),
            pltpu.VMEM((S * B, 3 * GH), f32),
            pltpu.VMEM((B * S, 512), f32),
            pltpu.VMEM((B * S, HID), f32),
            pltpu.VMEM((B * S, HID), f32),
            pltpu.VMEM((B * S, HID), f32),
            pltpu.VMEM((B, S, S), f32),
            pltpu.VMEM((S, B, S), f32),
            pltpu.VMEM((B * S, 3 * GH), f32),
            pltpu.VMEM((S, B, 3 * GH), f32),
        ],
        compiler_params=pltpu.CompilerParams(
            vmem_limit_bytes=56 * 1024 * 1024),
        name="seq_decode",
    )(*args)

    n_tile = 3200
    scinfo = pl.pallas_call(
        _proj_kernel,
        out_shape=jax.ShapeDtypeStruct((B * S, ALLKEY), f32),
        grid=(ALLKEY // n_tile,),
        in_specs=[
            pl.BlockSpec((S, B, 2 * GH), lambda i: (0, 0, 0)),
            pl.BlockSpec((n_tile, 2 * GH), lambda i: (i, 0)),
            pl.BlockSpec((1, n_tile), lambda i: (0, i)),
        ],
        out_specs=pl.BlockSpec((B * S, n_tile), lambda i: (0, i)),
        scratch_shapes=[pltpu.VMEM((B * S, 2 * GH), f32)],
        compiler_params=pltpu.CompilerParams(
            dimension_semantics=("arbitrary",),
            vmem_limit_bytes=56 * 1024 * 1024),
        name="out_proj",
    )(endt, Wout, bout.reshape(1, ALLKEY))

    return scinfo.reshape(B, S, ALLKEY)


# speedup vs baseline: 1.4203x; 1.0015x over previous
import jax
import jax.numpy as jnp
from jax.experimental import pallas as pl
from jax.experimental.pallas import tpu as pltpu

ALLKEY = 32000
HID = 150
GH = 256
B = 16
S = 48


def _gates(gx, gh, h):
    r = jax.nn.sigmoid(gx[:, 0:GH] + gh[:, 0:GH])
    z = jax.nn.sigmoid(gx[:, GH:2 * GH] + gh[:, GH:2 * GH])
    n = jnp.tanh(gx[:, 2 * GH:3 * GH] + r * gh[:, 2 * GH:3 * GH])
    return (1.0 - z) * n + z * h


def _dot(a, b):
    return jnp.dot(a, b, preferred_element_type=jnp.float32)


def _seq_kernel(
    dt_idx_ref, et_idx_ref, emb_ref,
    w0f_x_ref, w0f_h_ref, b0f_x_ref, b0f_h_ref,
    w0b_x_ref, w0b_h_ref, b0b_x_ref, b0b_h_ref,
    w1f_x_ref, w1f_h_ref, b1f_x_ref, b1f_h_ref,
    w1b_x_ref, w1b_h_ref, b1b_x_ref, b1b_h_ref,
    wd0_x_ref, w0cat_ref, wbig_ref, bd0_x_ref, bd0_h_ref,
    bd1_x_ref, bd1_h_ref,
    wo_ref, bo_ref, wq_ref, wk_ref, wv_ref,
    cw_ref, cb_ref,
    endt_ref,
    dtf_ref, etf_ref, sem_dt, sem_et,
    gx0f_ref, gx0b_ref, x1_ref, gx1f_ref, gx1b_ref,
    ko_ref, q_ref, k_ref, v_ref,
    expa_ref, expal_ref, vw_ref, gxs_ref,
):
    f32 = jnp.float32
    zeros_h = jnp.zeros((B, GH), f32)

    for r in range(S * B):
        t, b = r // B, r % B
        pltpu.make_async_copy(
            emb_ref.at[dt_idx_ref[b * S + t]], dtf_ref.at[r], sem_dt).start()
    for r in range(B * S):
        if r % S == 0:
            etf_ref[pl.ds(r, 1), :] = jnp.zeros((1, HID), f32)
        else:
            pltpu.make_async_copy(
                emb_ref.at[et_idx_ref[r - 1]], etf_ref.at[r], sem_et).start()
    for r in range(S * B):
        t, b = r // B, r % B
        pltpu.make_async_copy(
            emb_ref.at[dt_idx_ref[b * S + t]], dtf_ref.at[r], sem_dt).wait()

    gx0f_ref[...] = _dot(dtf_ref[...], w0f_x_ref[...]) + b0f_x_ref[...]
    gx0b_ref[...] = _dot(dtf_ref[...], w0b_x_ref[...]) + b0b_x_ref[...]

    def enc0_body(t, carry):
        hf, hb = carry
        gxf = gx0f_ref[pl.ds(t * B, B), :]
        ghf = _dot(hf, w0f_h_ref[...]) + b0f_h_ref[...]
        hf = _gates(gxf, ghf, hf)
        x1_ref[t, :, 0:GH] = hf
        tb = (S - 1) - t
        gxb = gx0b_ref[pl.ds(tb * B, B), :]
        ghb = _dot(hb, w0b_h_ref[...]) + b0b_h_ref[...]
        hb = _gates(gxb, ghb, hb)
        x1_ref[tb, :, GH:2 * GH] = hb
        return (hf, hb)

    hf0, hb0 = jax.lax.fori_loop(0, S, enc0_body, (zeros_h, zeros_h))

    x1_flat = x1_ref[...].reshape(S * B, 2 * GH)
    gx1f_ref[...] = _dot(x1_flat, w1f_x_ref[...]) + b1f_x_ref[...]
    gx1b_ref[...] = _dot(x1_flat, w1b_x_ref[...]) + b1b_x_ref[...]

    def enc1_body(t, carry):
        hf, hb = carry
        gxf = gx1f_ref[pl.ds(t * B, B), :]
        hf = _gates(gxf, _dot(hf, w1f_h_ref[...]) + b1f_h_ref[...], hf)
        tb = (S - 1) - t
        gxb = gx1b_ref[pl.ds(tb * B, B), :]
        hb = _gates(gxb, _dot(hb, w1b_h_ref[...]) + b1b_h_ref[...], hb)
        return (hf, hb)

    hf1, hb1 = jax.lax.fori_loop(0, S, enc1_body, (zeros_h, zeros_h))

    h0 = (cw_ref[0, 0] * hf0 + cw_ref[0, 1] * hb0
          + cw_ref[0, 2] * hf1 + cw_ref[0, 3] * hb1 + cb_ref[0])
    h1 = (cw_ref[1, 0] * hf0 + cw_ref[1, 1] * hb0
          + cw_ref[1, 2] * hf1 + cw_ref[1, 3] * hb1 + cb_ref[1])

    for r in range(B * S):
        if r % S != 0:
            pltpu.make_async_copy(
                emb_ref.at[et_idx_ref[r - 1]], etf_ref.at[r], sem_et).wait()
    ko_ref[...] = _dot(etf_ref[...], wo_ref[...]) + bo_ref[...]
    ko = ko_ref[...]
    q_ref[...] = _dot(ko, wq_ref[...])
    k_ref[...] = _dot(ko, wk_ref[...])
    v_ref[...] = _dot(ko, wv_ref[...])
    vw_ref[...] = _dot(v_ref[...], wd0_x_ref[...]).astype(jnp.bfloat16)

    for b in range(B):
        qb = q_ref[pl.ds(b * S, S), :]
        kb = k_ref[pl.ds(b * S, S), :]
        ab = jax.lax.dot_general(qb, kb, (((1,), (1,)), ((), ())),
                                 preferred_element_type=f32)
        ea = jnp.exp(ab)
        expa_ref[b] = ea
        expal_ref[:, b, :] = ea

    lane_m = jax.lax.broadcasted_iota(jnp.int32, (B, S), 1)
    b0x = bd0_x_ref[...]
    b0h = bd0_h_ref[...]
    b1x = bd1_x_ref[...]
    b1h = bd1_h_ref[...]

    def fused_parts(h0, h1):
        big0 = _dot(h0.astype(jnp.bfloat16), w0cat_ref[...])
        big1 = _dot(h1.astype(jnp.bfloat16), wbig_ref[...])
        gh0 = big0[:, 0:3 * GH] + b0h
        gx1 = big0[:, 3 * GH:6 * GH] + b1x
        gh1 = big1 + b1h
        return gh0, gh1, gx1

    def outer_body(idx, carry):
        h0, h1, den = carry
        den = den + expal_ref[idx]
        scale = jnp.where(lane_m <= idx, 1.0 / den, 0.0)

        for b in range(B):
            pb = (expa_ref[b] * scale[b:b + 1, :]).astype(jnp.bfloat16)
            gxs_ref[:, b, :] = _dot(pb, vw_ref[pl.ds(b * S, S), :]) + b0x

        def inner_body(l, hh):
            h0, h1 = hh
            gh0, gh1, gx1 = fused_parts(h0, h1)
            new_h0 = _gates(gxs_ref[l], gh0, h0)
            new_h1 = _gates(gx1, gh1, h1)
            h1 = jnp.where(l > 0, new_h1, h1)
            return (new_h0, h1)

        h0, h1 = jax.lax.fori_loop(0, idx + 1, inner_body, (h0, h1))
        _, gh1, gx1 = fused_parts(h0, h1)
        h1 = _gates(gx1, gh1, h1)
        endt_ref[idx, :, 0:GH] = h0
        endt_ref[idx, :, GH:2 * GH] = h1
        return (h0, h1, den)

    jax.lax.fori_loop(0, S, outer_body, (h0, h1, jnp.zeros((B, S), f32)))


def _proj_kernel(a_ref, w_ref, b_ref, o_ref, a2_ref):
    @pl.when(pl.program_id(0) == 0)
    def _():
        a2_ref[...] = jnp.swapaxes(a_ref[...], 0, 1).reshape(B * S, 2 * GH)

    o_ref[...] = jax.lax.dot_general(
        a2_ref[...], w_ref[...], (((1,), (1,)), ((), ())),
        preferred_element_type=jnp.float32) + b_ref[...]


def kernel(x, target, emb, enc0f, enc0b, enc1f, enc1b, dec0, dec1,
           conv_w, conv_b, lin_o, lin_out, linQ, linK, linV):
    f32 = jnp.float32
    dt_idx = x.reshape(B * S)
    et_idx = target.reshape(B * S)

    def gru_w(p):
        wih, whh, bih, bhh = p
        return (wih.T, whh.T, bih.reshape(1, -1), bhh.reshape(1, -1))

    Wo, bo = lin_o
    Wout, bout = lin_out
    WQ, _bQ = linQ
    WK, _bK = linK
    WV, _bV = linV

    wih0, whh0, bih0, bhh0 = dec0
    wih1, whh1, bih1, bhh1 = dec1
    bf16 = jnp.bfloat16
    w0cat = jnp.concatenate([whh0.T, wih1.T], axis=1).astype(bf16)
    wbig = whh1.T.astype(bf16)

    args = (
        dt_idx, et_idx, emb,
        *gru_w(enc0f), *gru_w(enc0b), *gru_w(enc1f), *gru_w(enc1b),
        wih0.T, w0cat, wbig, bih0.reshape(1, -1), bhh0.reshape(1, -1),
        bih1.reshape(1, -1), bhh1.reshape(1, -1),
        Wo.T, bo.reshape(1, -1), WQ.T, WK.T, WV.T,
        conv_w[:, :, 0], conv_b,
    )
    n_in = len(args)
    in_specs = [
        pl.BlockSpec(memory_space=pltpu.SMEM),
        pl.BlockSpec(memory_space=pltpu.SMEM),
        pl.BlockSpec(memory_space=pl.ANY),
    ] + [pl.BlockSpec(memory_space=pltpu.VMEM)] * (n_in - 5) + [
        pl.BlockSpec(memory_space=pltpu.SMEM),
        pl.BlockSpec(memory_space=pltpu.SMEM),
    ]

    endt = pl.pallas_call(
        _seq_kernel,
        out_shape=jax.ShapeDtypeStruct((S, B, 2 * GH), f32),
        in_specs=in_specs,
        out_specs=pl.BlockSpec(memory_space=pltpu.VMEM),
        scratch_shapes=[
            pltpu.VMEM((S * B, HID), f32),
            pltpu.VMEM((B * S, HID), f32),
            pltpu.SemaphoreType.DMA,
            pltpu.SemaphoreType.DMA,
            pltpu.VMEM((S * B, 3 * GH), f32),
            pltpu.VMEM((S * B, 3 * GH), f32),
            pltpu.VMEM((S, B, 2 * GH), f32),
            pltpu.VMEM((S * B, 3 * GH), f32),
            pltpu.VMEM((S * B, 3 * GH), f32),
            pltpu.VMEM((B * S, 512), f32),
            pltpu.VMEM((B * S, HID), f32),
            pltpu.VMEM((B * S, HID), f32),
            pltpu.VMEM((B * S, HID), f32),
            pltpu.VMEM((B, S, S), f32),
            pltpu.VMEM((S, B, S), f32),
            pltpu.VMEM((B * S, 3 * GH), jnp.bfloat16),
            pltpu.VMEM((S, B, 3 * GH), f32),
        ],
        compiler_params=pltpu.CompilerParams(
            vmem_limit_bytes=56 * 1024 * 1024),
        name="seq_decode",
    )(*args)

    n_tile = 3200
    scinfo = pl.pallas_call(
        _proj_kernel,
        out_shape=jax.ShapeDtypeStruct((B * S, ALLKEY), f32),
        grid=(ALLKEY // n_tile,),
        in_specs=[
            pl.BlockSpec((S, B, 2 * GH), lambda i: (0, 0, 0)),
            pl.BlockSpec((n_tile, 2 * GH), lambda i: (i, 0)),
            pl.BlockSpec((1, n_tile), lambda i: (0, i)),
        ],
        out_specs=pl.BlockSpec((B * S, n_tile), lambda i: (0, i)),
        scratch_shapes=[pltpu.VMEM((B * S, 2 * GH), f32)],
        compiler_params=pltpu.CompilerParams(
            dimension_semantics=("arbitrary",),
            vmem_limit_bytes=56 * 1024 * 1024),
        name="out_proj",
    )(endt, Wout, bout.reshape(1, ALLKEY))

    return scinfo.reshape(B, S, ALLKEY)
